# Initial kernel scaffold; baseline (speedup 1.0000x reference)
import jax
import jax.numpy as jnp
from jax.experimental import pallas as pl


def kernel(table, num_group):
    raise NotImplementedError("write your pallas kernel here")


# speedup vs baseline: 5.8303x; 5.8303x over previous
import jax
import jax.numpy as jnp
from jax import lax
from jax.experimental import pallas as pl
from jax.experimental.pallas import tpu as pltpu

IDX_BLOCK = 2048
UNROLL = 16


def _gather_kernel(idx_ref, table_ref, out_ref):
    def body(c, carry):
        base = c * UNROLL
        for u in range(UNROLL):
            j = base + u
            out_ref[j] = table_ref[idx_ref[0, 0, j]]
        return carry

    lax.fori_loop(0, IDX_BLOCK // UNROLL, body, 0)


def _gather(table: jax.Array, flat_idx: jax.Array) -> jax.Array:
    num_rows, dim = table.shape
    n = flat_idx.shape[0]
    n_pad = ((n + IDX_BLOCK - 1) // IDX_BLOCK) * IDX_BLOCK
    num_blocks = n_pad // IDX_BLOCK
    idx3d = jnp.pad(flat_idx, (0, n_pad - n)).reshape(num_blocks, 1, IDX_BLOCK)
    table3 = table.reshape(num_rows, 1, dim)

    itemsize = table.dtype.itemsize
    cost = pl.CostEstimate(
        flops=0,
        transcendentals=0,
        bytes_accessed=num_rows * dim * itemsize + n_pad * dim * itemsize
        + n_pad * 4,
    )

    out = pl.pallas_call(
        _gather_kernel,
        grid=(num_blocks,),
        in_specs=[
            pl.BlockSpec((1, 1, IDX_BLOCK), lambda i: (i, 0, 0),
                         memory_space=pltpu.SMEM),
            pl.BlockSpec((num_rows, 1, dim), lambda i: (0, 0, 0)),
        ],
        out_specs=pl.BlockSpec((IDX_BLOCK, 1, dim), lambda i: (i, 0, 0)),
        out_shape=jax.ShapeDtypeStruct((n_pad, 1, dim), table.dtype),
        compiler_params=pltpu.CompilerParams(
            dimension_semantics=("parallel",),
            vmem_limit_bytes=32 * 1024 * 1024,
        ),
        cost_estimate=cost,
    )(idx3d, table3)

    return out.reshape(n_pad, dim)[:n]


def kernel(table, num_group):
    num_rows, dim = table.shape
    flat_idx = jnp.clip(num_group.reshape(-1).astype(jnp.int32), 0, num_rows - 1)
    out = _gather(table, flat_idx)
    return out.reshape(num_group.shape + (dim,))


# speedup vs baseline: 6.8107x; 1.1682x over previous
import jax
import jax.numpy as jnp
from jax import lax
from jax.experimental import pallas as pl
from jax.experimental.pallas import tpu as pltpu

IDX_BLOCK = 4096
UNROLL = 32


def _gather_kernel(idx_ref, table_ref, out_ref):
    def body(c, carry):
        base = c * UNROLL
        rows = [table_ref[idx_ref[0, 0, base + u]] for u in range(UNROLL)]
        for u in range(UNROLL):
            out_ref[base + u] = rows[u]
        return carry

    lax.fori_loop(0, IDX_BLOCK // UNROLL, body, 0)


def _gather(table: jax.Array, flat_idx: jax.Array) -> jax.Array:
    num_rows, dim = table.shape
    n = flat_idx.shape[0]
    n_pad = ((n + IDX_BLOCK - 1) // IDX_BLOCK) * IDX_BLOCK
    num_blocks = n_pad // IDX_BLOCK
    idx3d = jnp.pad(flat_idx, (0, n_pad - n)).reshape(num_blocks, 1, IDX_BLOCK)
    table3 = table.reshape(num_rows, 1, dim)

    itemsize = table.dtype.itemsize
    cost = pl.CostEstimate(
        flops=0,
        transcendentals=0,
        bytes_accessed=num_rows * dim * itemsize + n_pad * dim * itemsize
        + n_pad * 4,
    )

    out = pl.pallas_call(
        _gather_kernel,
        grid=(num_blocks,),
        in_specs=[
            pl.BlockSpec((1, 1, IDX_BLOCK), lambda i: (i, 0, 0),
                         memory_space=pltpu.SMEM),
            pl.BlockSpec((num_rows, 1, dim), lambda i: (0, 0, 0)),
        ],
        out_specs=pl.BlockSpec((IDX_BLOCK, 1, dim), lambda i: (i, 0, 0)),
        out_shape=jax.ShapeDtypeStruct((n_pad, 1, dim), table.dtype),
        compiler_params=pltpu.CompilerParams(
            dimension_semantics=("parallel",),
            vmem_limit_bytes=32 * 1024 * 1024,
        ),
        cost_estimate=cost,
    )(idx3d, table3)

    return out.reshape(n_pad, dim)[:n]


def kernel(table, num_group):
    num_rows, dim = table.shape
    flat_idx = jnp.clip(num_group.reshape(-1).astype(jnp.int32), 0, num_rows - 1)
    out = _gather(table, flat_idx)
    return out.reshape(num_group.shape + (dim,))


# speedup vs baseline: 11.5096x; 1.6899x over previous
import numpy as np

import jax
import jax.numpy as jnp
from jax import lax
from jax.experimental import pallas as pl
from jax.experimental.pallas import tpu as pltpu
from jax.experimental.shard_map import shard_map
from jax.sharding import Mesh, PartitionSpec as P

IDX_BLOCK = 4096
UNROLL = 32


def _gather_kernel(idx_ref, table_ref, out_ref):
    def body(c, carry):
        base = c * UNROLL
        rows = [table_ref[idx_ref[0, 0, base + u]] for u in range(UNROLL)]
        for u in range(UNROLL):
            out_ref[base + u] = rows[u]
        return carry

    lax.fori_loop(0, IDX_BLOCK // UNROLL, body, 0)


def _gather(table: jax.Array, flat_idx: jax.Array) -> jax.Array:
    num_rows, dim = table.shape
    n = flat_idx.shape[0]
    n_pad = ((n + IDX_BLOCK - 1) // IDX_BLOCK) * IDX_BLOCK
    num_blocks = n_pad // IDX_BLOCK
    idx3d = jnp.pad(flat_idx, (0, n_pad - n)).reshape(num_blocks, 1, IDX_BLOCK)
    table3 = table.reshape(num_rows, 1, dim)

    itemsize = table.dtype.itemsize
    cost = pl.CostEstimate(
        flops=0,
        transcendentals=0,
        bytes_accessed=num_rows * dim * itemsize + n_pad * dim * itemsize
        + n_pad * 4,
    )

    out = pl.pallas_call(
        _gather_kernel,
        grid=(num_blocks,),
        in_specs=[
            pl.BlockSpec((1, 1, IDX_BLOCK), lambda i: (i, 0, 0),
                         memory_space=pltpu.SMEM),
            pl.BlockSpec((num_rows, 1, dim), lambda i: (0, 0, 0)),
        ],
        out_specs=pl.BlockSpec((IDX_BLOCK, 1, dim), lambda i: (i, 0, 0)),
        out_shape=jax.ShapeDtypeStruct((n_pad, 1, dim), table.dtype),
        compiler_params=pltpu.CompilerParams(
            dimension_semantics=("parallel",),
            vmem_limit_bytes=32 * 1024 * 1024,
        ),
        cost_estimate=cost,
    )(idx3d, table3)

    return out.reshape(n_pad, dim)[:n]


def kernel(table, num_group):
    num_rows, dim = table.shape
    flat_idx = jnp.clip(num_group.reshape(-1).astype(jnp.int32), 0, num_rows - 1)
    n = flat_idx.shape[0]

    devs = jax.devices()
    if len(devs) >= 2 and n % (2 * IDX_BLOCK) == 0:
        mesh = Mesh(np.array(devs[:2]), ("x",))
        out = shard_map(
            _gather, mesh=mesh,
            in_specs=(P(), P("x")), out_specs=P("x"), check_rep=False,
        )(table, flat_idx)
    else:
        out = _gather(table, flat_idx)
    return out.reshape(num_group.shape + (dim,))


# speedup vs baseline: 11.5556x; 1.0040x over previous
import numpy as np

import jax
import jax.numpy as jnp
from jax import lax
from jax.experimental import pallas as pl
from jax.experimental.pallas import tpu as pltpu
from jax.experimental.shard_map import shard_map
from jax.sharding import Mesh, PartitionSpec as P

IDX_BLOCK = 4096
UNROLL = 32


def _gather_kernel(idx_ref, table_ref, out_ref):
    def body(c, carry):
        base = c * UNROLL
        rows = [table_ref[idx_ref[0, 0, base + u]] for u in range(UNROLL)]
        for u in range(UNROLL):
            out_ref[base + u] = rows[u]
        return carry

    lax.fori_loop(0, IDX_BLOCK // UNROLL, body, 0)


def _gather(table: jax.Array, flat_idx: jax.Array) -> jax.Array:
    num_rows, dim = table.shape
    n = flat_idx.shape[0]
    n_pad = ((n + IDX_BLOCK - 1) // IDX_BLOCK) * IDX_BLOCK
    num_blocks = n_pad // IDX_BLOCK
    idx3d = jnp.pad(flat_idx, (0, n_pad - n)).reshape(num_blocks, 1, IDX_BLOCK)
    table3 = table.reshape(num_rows, 1, dim)

    itemsize = table.dtype.itemsize
    cost = pl.CostEstimate(
        flops=0,
        transcendentals=0,
        bytes_accessed=num_rows * dim * itemsize + n_pad * dim * itemsize
        + n_pad * 4,
    )

    out = pl.pallas_call(
        _gather_kernel,
        grid=(num_blocks,),
        in_specs=[
            pl.BlockSpec((1, 1, IDX_BLOCK), lambda i: (i, 0, 0),
                         memory_space=pltpu.SMEM),
            pl.BlockSpec((num_rows, 1, dim), lambda i: (0, 0, 0)),
        ],
        out_specs=pl.BlockSpec((IDX_BLOCK, 1, dim), lambda i: (i, 0, 0)),
        out_shape=jax.ShapeDtypeStruct((n_pad, 1, dim), table.dtype),
        compiler_params=pltpu.CompilerParams(
            dimension_semantics=("parallel",),
            vmem_limit_bytes=32 * 1024 * 1024,
        ),
        cost_estimate=cost,
    )(idx3d, table3)

    return out.reshape(n_pad, dim)[:n]


def kernel(table, num_group):
    num_rows, dim = table.shape
    flat_idx = jnp.clip(num_group.reshape(-1).astype(jnp.int32), 0, num_rows - 1)
    n = flat_idx.shape[0]

    devs = jax.devices()
    if len(devs) >= 2 and n % (2 * IDX_BLOCK) == 0 and num_rows <= 32768:
        idx16 = flat_idx.astype(jnp.int16)
        mesh = Mesh(np.array(devs[:2]), ("x",))
        out = shard_map(
            lambda t, i: _gather(t, i.astype(jnp.int32)), mesh=mesh,
            in_specs=(P(), P("x")), out_specs=P("x"), check_rep=False,
        )(table, idx16)
    else:
        out = _gather(table, flat_idx)
    return out.reshape(num_group.shape + (dim,))
